# baseline (speedup 1.0000x reference)
import functools

import jax
import jax.numpy as jnp
from jax import lax
from jax.experimental import pallas as pl
from jax.experimental.pallas import tpu as pltpu
from jax.experimental.pallas import tpu_sc as plsc

EMBEDDING_DIM = 32

_info = plsc.get_sparse_core_info()
_NC, _NS = _info.num_cores, _info.num_subcores
_NW = _NC * _NS


@functools.lru_cache(maxsize=None)
def _build_repack(n_rows, dim, chunk):
    assert n_rows % chunk == 0 and chunk % 8 == 0 and dim % 16 == 0
    n_chunks = n_rows // chunk
    trips = (n_chunks + _NW - 1) // _NW
    mesh = plsc.VectorSubcoreMesh(core_axis_name="c", subcore_axis_name="s")

    @functools.partial(
        pl.kernel,
        mesh=mesh,
        out_type=jax.ShapeDtypeStruct((n_rows * dim,), jnp.float32),
        compiler_params=pltpu.CompilerParams(use_tc_tiling_on_sc=True),
        scratch_types=(
            [pltpu.VMEM((chunk, dim), jnp.float32) for _ in range(2)]
            + [pltpu.VMEM((chunk * dim,), jnp.float32) for _ in range(2)]
            + [pltpu.SemaphoreType.DMA for _ in range(4)]
        ),
    )
    def k(w_hbm, out_hbm, gb0, gb1, ob0, ob1, isem0, isem1, osem0, osem1):
        gbufs = (gb0, gb1)
        obufs = (ob0, ob1)
        isems = (isem0, isem1)
        osems = (osem0, osem1)
        wid = lax.axis_index("s") * _NC + lax.axis_index("c")

        full = n_chunks // _NW
        rem = n_chunks - full * _NW

        in_h = [None] * full
        out_h = [None] * full

        def cid(g):
            return wid + _NW * g

        def start_in(g):
            s = g % 2
            r0 = pl.multiple_of(cid(g) * chunk, chunk)
            in_h[g] = pltpu.async_copy(
                w_hbm.at[pl.ds(r0, chunk), :], gbufs[s], isems[s])

        def repack(gb, ob):
            u = 5

            def body(i, carry):
                for q in range(u):
                    j = i * u + q
                    for h in range(dim // 16):
                        ob[pl.ds(j * dim + h * 16, 16)] = (
                            gb[j, pl.ds(h * 16, 16)])
                return carry
            lax.fori_loop(0, chunk // u, body, 0)

        def start_out(g):
            s = g % 2
            e0 = pl.multiple_of(cid(g) * (chunk * dim), chunk * dim)
            out_h[g] = pltpu.async_copy(
                obufs[s], out_hbm.at[pl.ds(e0, chunk * dim)], osems[s])

        if full > 0:
            start_in(0)
        for g in range(full):
            s = g % 2
            if g + 1 < full:
                start_in(g + 1)
            in_h[g].wait()
            repack(gbufs[s], obufs[s])
            if g - 2 >= 0:
                out_h[g - 2].wait()
            start_out(g)
        for g in range(max(0, full - 2), full):
            out_h[g].wait()

        if rem:
            def tail():
                r0 = pl.multiple_of((wid + _NW * full) * chunk, chunk)
                hi = pltpu.async_copy(
                    w_hbm.at[pl.ds(r0, chunk), :], gbufs[0], isems[0])
                hi.wait()
                repack(gbufs[0], obufs[0])
                e0 = pl.multiple_of(
                    (wid + _NW * full) * (chunk * dim), chunk * dim)
                ho = pltpu.async_copy(
                    obufs[0], out_hbm.at[pl.ds(e0, chunk * dim)], osems[0])
                ho.wait()
            pl.when(wid < rem)(tail)

    return k


@functools.lru_cache(maxsize=None)
def _build_gather(total, dim, chunk, nbuf, inflight):
    assert total % _NW == 0
    b_per_w = total // _NW
    assert b_per_w % chunk == 0
    n_chunks = b_per_w // chunk
    assert inflight < nbuf
    mesh = plsc.VectorSubcoreMesh(core_axis_name="c", subcore_axis_name="s")

    @functools.partial(
        pl.kernel,
        mesh=mesh,
        out_type=jax.ShapeDtypeStruct((total, dim), jnp.float32),
        compiler_params=pltpu.CompilerParams(use_tc_tiling_on_sc=False),
        scratch_types=(
            [pltpu.VMEM((b_per_w,), jnp.int32)]
            + [pltpu.VMEM((chunk, dim), jnp.float32) for _ in range(nbuf)]
            + [pltpu.SemaphoreType.DMA for _ in range(2 * nbuf)]
        ),
    )
    def k(table_hbm, idx_hbm, out_hbm, idx_v, *rest):
        bufs = rest[:nbuf]
        gsems = rest[nbuf:2 * nbuf]
        osems = rest[2 * nbuf:]
        wid = lax.axis_index("s") * _NC + lax.axis_index("c")
        base = wid * b_per_w
        pltpu.sync_copy(idx_hbm.at[pl.ds(base, b_per_w)], idx_v)

        gather_h = [None] * n_chunks
        out_h = [None] * n_chunks

        def start_gather(c):
            s = c % nbuf
            gather_h[c] = pltpu.async_copy(
                table_hbm.at[idx_v.at[pl.ds(c * chunk, chunk)]],
                bufs[s], gsems[s])

        for j in range(min(inflight, n_chunks)):
            start_gather(j)
        for c in range(n_chunks):
            f = c + inflight
            if f < n_chunks:
                prev = f - nbuf
                if prev >= 0:
                    out_h[prev].wait()
                start_gather(f)
            gather_h[c].wait()
            s = c % nbuf
            out_h[c] = pltpu.async_copy(
                bufs[s], out_hbm.at[pl.ds(base + c * chunk, chunk)],
                osems[s])
        for c in range(max(0, n_chunks - nbuf), n_chunks):
            out_h[c].wait()

    return k


def kernel(token_ids, weight):
    batch, fields = token_ids.shape
    n_rows, dim = weight.shape
    total = batch * fields
    flat_idx = token_ids.reshape(total).astype(jnp.int32)
    packed = _build_repack(n_rows, dim, 320)(weight)
    table = packed.reshape(n_rows, dim)
    out = _build_gather(total, dim, 416, 8, 6)(table, flat_idx)
    return out.reshape(batch, fields, dim)


# speedup vs baseline: 1.0597x; 1.0597x over previous
import functools

import jax
import jax.numpy as jnp
from jax import lax
from jax.experimental import pallas as pl
from jax.experimental.pallas import tpu as pltpu
from jax.experimental.pallas import tpu_sc as plsc

EMBEDDING_DIM = 32

_info = plsc.get_sparse_core_info()
_NC, _NS = _info.num_cores, _info.num_subcores
_NW = _NC * _NS


@functools.lru_cache(maxsize=None)
def _build_repack(n_rows, dim, chunk):
    assert n_rows % chunk == 0 and chunk % 8 == 0 and dim % 16 == 0
    n_chunks = n_rows // chunk
    trips = (n_chunks + _NW - 1) // _NW
    mesh = plsc.VectorSubcoreMesh(core_axis_name="c", subcore_axis_name="s")

    @functools.partial(
        pl.kernel,
        mesh=mesh,
        out_type=jax.ShapeDtypeStruct((n_rows * dim,), jnp.float32),
        compiler_params=pltpu.CompilerParams(use_tc_tiling_on_sc=True),
        scratch_types=(
            [pltpu.VMEM((chunk, dim), jnp.float32) for _ in range(2)]
            + [pltpu.VMEM((chunk * dim,), jnp.float32) for _ in range(2)]
            + [pltpu.SemaphoreType.DMA for _ in range(4)]
        ),
    )
    def k(w_hbm, out_hbm, gb0, gb1, ob0, ob1, isem0, isem1, osem0, osem1):
        gbufs = (gb0, gb1)
        obufs = (ob0, ob1)
        isems = (isem0, isem1)
        osems = (osem0, osem1)
        wid = lax.axis_index("s") * _NC + lax.axis_index("c")

        full = n_chunks // _NW
        rem = n_chunks - full * _NW

        in_h = [None] * full
        out_h = [None] * full

        def cid(g):
            return wid + _NW * g

        def start_in(g):
            s = g % 2
            r0 = pl.multiple_of(cid(g) * chunk, chunk)
            in_h[g] = pltpu.async_copy(
                w_hbm.at[pl.ds(r0, chunk), :], gbufs[s], isems[s])

        def repack(gb, ob):
            u = 5

            def body(i, carry):
                for q in range(u):
                    j = i * u + q
                    for h in range(dim // 16):
                        ob[pl.ds(j * dim + h * 16, 16)] = (
                            gb[j, pl.ds(h * 16, 16)])
                return carry
            lax.fori_loop(0, chunk // u, body, 0)

        def start_out(g):
            s = g % 2
            e0 = pl.multiple_of(cid(g) * (chunk * dim), chunk * dim)
            out_h[g] = pltpu.async_copy(
                obufs[s], out_hbm.at[pl.ds(e0, chunk * dim)], osems[s])

        if full > 0:
            start_in(0)
        for g in range(full):
            s = g % 2
            if g + 1 < full:
                start_in(g + 1)
            in_h[g].wait()
            repack(gbufs[s], obufs[s])
            if g - 2 >= 0:
                out_h[g - 2].wait()
            start_out(g)
        for g in range(max(0, full - 2), full):
            out_h[g].wait()

        if rem:
            def tail():
                r0 = pl.multiple_of((wid + _NW * full) * chunk, chunk)
                hi = pltpu.async_copy(
                    w_hbm.at[pl.ds(r0, chunk), :], gbufs[0], isems[0])
                hi.wait()
                repack(gbufs[0], obufs[0])
                e0 = pl.multiple_of(
                    (wid + _NW * full) * (chunk * dim), chunk * dim)
                ho = pltpu.async_copy(
                    obufs[0], out_hbm.at[pl.ds(e0, chunk * dim)], osems[0])
                ho.wait()
            pl.when(wid < rem)(tail)

    return k


@functools.lru_cache(maxsize=None)
def _build_gather(total, dim, chunk, nbuf, inflight):
    assert total % _NW == 0
    b_per_w = total // _NW
    assert b_per_w % chunk == 0
    n_chunks = b_per_w // chunk
    assert inflight < nbuf
    mesh = plsc.VectorSubcoreMesh(core_axis_name="c", subcore_axis_name="s")

    @functools.partial(
        pl.kernel,
        mesh=mesh,
        out_type=jax.ShapeDtypeStruct((total, dim), jnp.float32),
        compiler_params=pltpu.CompilerParams(use_tc_tiling_on_sc=False),
        scratch_types=(
            [pltpu.VMEM((b_per_w,), jnp.int32)]
            + [pltpu.VMEM((chunk, dim), jnp.float32) for _ in range(nbuf)]
            + [pltpu.SemaphoreType.DMA for _ in range(2 * nbuf)]
        ),
    )
    def k(table_hbm, idx_hbm, out_hbm, idx_v, *rest):
        bufs = rest[:nbuf]
        gsems = rest[nbuf:2 * nbuf]
        osems = rest[2 * nbuf:]
        wid = lax.axis_index("s") * _NC + lax.axis_index("c")
        base = wid * b_per_w
        pltpu.sync_copy(idx_hbm.at[pl.ds(base, b_per_w)], idx_v)

        gather_h = [None] * n_chunks
        out_h = [None] * n_chunks

        def start_gather(c):
            s = c % nbuf
            gather_h[c] = pltpu.async_copy(
                table_hbm.at[idx_v.at[pl.ds(c * chunk, chunk)]],
                bufs[s], gsems[s])

        for j in range(min(inflight, n_chunks)):
            start_gather(j)
        for c in range(n_chunks):
            f = c + inflight
            if f < n_chunks:
                prev = f - nbuf
                if prev >= 0:
                    out_h[prev].wait()
                start_gather(f)
            gather_h[c].wait()
            s = c % nbuf
            out_h[c] = pltpu.async_copy(
                bufs[s], out_hbm.at[pl.ds(base + c * chunk, chunk)],
                osems[s])
        for c in range(max(0, n_chunks - nbuf), n_chunks):
            out_h[c].wait()

    return k


def kernel(token_ids, weight):
    batch, fields = token_ids.shape
    n_rows, dim = weight.shape
    total = batch * fields
    flat_idx = token_ids.reshape(total).astype(jnp.int32)
    out = _build_gather(total, dim, 416, 8, 6)(weight, flat_idx)
    return out.reshape(batch, fields, dim)


# speedup vs baseline: 1.1244x; 1.0610x over previous
import functools

import jax
import jax.numpy as jnp
from jax import lax
from jax.experimental import pallas as pl
from jax.experimental.pallas import tpu as pltpu
from jax.experimental.pallas import tpu_sc as plsc

EMBEDDING_DIM = 32

_info = plsc.get_sparse_core_info()
_NC, _NS = _info.num_cores, _info.num_subcores
_NW = _NC * _NS


@functools.lru_cache(maxsize=None)
def _build_gather(fields, batch, dim, nbuf, inflight):
    assert batch % _NW == 0
    b_per_w = batch // _NW
    chunk = b_per_w
    n_chunks = fields
    assert inflight < nbuf
    mesh = plsc.VectorSubcoreMesh(core_axis_name="c", subcore_axis_name="s")

    @functools.partial(
        pl.kernel,
        mesh=mesh,
        out_type=jax.ShapeDtypeStruct((fields * batch, dim), jnp.float32),
        compiler_params=pltpu.CompilerParams(use_tc_tiling_on_sc=False),
        scratch_types=(
            [pltpu.VMEM((fields * b_per_w,), jnp.int32)]
            + [pltpu.VMEM((chunk, dim), jnp.float32) for _ in range(nbuf)]
            + [pltpu.SemaphoreType.DMA for _ in range(2 * nbuf + 1)]
        ),
    )
    def k(table_hbm, idx_hbm, out_hbm, idx_v, *rest):
        bufs = rest[:nbuf]
        gsems = rest[nbuf:2 * nbuf]
        osems = rest[2 * nbuf:3 * nbuf]
        isem = rest[3 * nbuf]
        wid = lax.axis_index("s") * _NC + lax.axis_index("c")
        b0 = pl.multiple_of(wid * b_per_w, b_per_w)

        idx_h = [None] * fields
        for f in range(fields):
            idx_h[f] = pltpu.async_copy(
                idx_hbm.at[f, pl.ds(b0, b_per_w)],
                idx_v.at[pl.ds(f * b_per_w, b_per_w)], isem)

        gather_h = [None] * n_chunks
        out_h = [None] * n_chunks

        def start_gather(c):
            s = c % nbuf
            gather_h[c] = pltpu.async_copy(
                table_hbm.at[idx_v.at[pl.ds(c * chunk, chunk)]],
                bufs[s], gsems[s])

        for f in range(fields):
            idx_h[f].wait()
        for j in range(min(inflight, n_chunks)):
            start_gather(j)
        for c in range(n_chunks):
            f = c + inflight
            if f < n_chunks:
                prev = f - nbuf
                if prev >= 0:
                    out_h[prev].wait()
                start_gather(f)
            gather_h[c].wait()
            s = c % nbuf
            out_h[c] = pltpu.async_copy(
                bufs[s], out_hbm.at[pl.ds(c * batch + b0, chunk)],
                osems[s])
        for c in range(max(0, n_chunks - nbuf), n_chunks):
            out_h[c].wait()

    return k


def kernel(token_ids, weight):
    batch, fields = token_ids.shape
    n_rows, dim = weight.shape
    idx_t = token_ids.T.astype(jnp.int32)
    out = _build_gather(fields, batch, dim, 6, 4)(weight, idx_t)
    return out.reshape(fields, batch, dim).transpose(1, 0, 2)
